# Initial kernel scaffold; baseline (speedup 1.0000x reference)
import jax
import jax.numpy as jnp
from jax.experimental import pallas as pl


def kernel(ids_1, ids_2, ids_3, patch_mask, emb_1, emb_2, emb_3):
    raise NotImplementedError("write your pallas kernel here")


# speedup vs baseline: 15.7667x; 15.7667x over previous
import functools

import jax
import jax.numpy as jnp
from jax import lax
from jax.experimental import pallas as pl
from jax.experimental.pallas import tpu as pltpu
from jax.experimental.pallas import tpu_sc as plsc

B, P, L = 1024, 50, 20
BUCKETS, D = 4096, 64
BP = B * P
NW = 32
PPW = BP // NW
C = 16
NCHUNK = PPW // C
IPC = C * L
G = 64
NG = IPC // G


def _sc_embed(ids1, ids2, ids3, maskf, emb_1, emb_2, emb_3):
    mesh = plsc.VectorSubcoreMesh(core_axis_name="c", subcore_axis_name="s")

    @functools.partial(
        pl.kernel,
        mesh=mesh,
        out_type=jax.ShapeDtypeStruct((BP, D), jnp.float32),
        compiler_params=pltpu.CompilerParams(use_tc_tiling_on_sc=False),
        scratch_types=[
            pltpu.VMEM((IPC,), jnp.int32),
            pltpu.VMEM((IPC,), jnp.int32),
            pltpu.VMEM((IPC,), jnp.int32),
            pltpu.VMEM((IPC, D), jnp.float32),
            pltpu.VMEM((IPC, D), jnp.float32),
            pltpu.VMEM((IPC, D), jnp.float32),
            pltpu.VMEM((C, D), jnp.float32),
            pltpu.VMEM((C,), jnp.float32),
            pltpu.SemaphoreType.DMA,
        ],
    )
    def k(ids1_hbm, ids2_hbm, ids3_hbm, mask_hbm, t1, t2, t3, out_hbm,
          idx1_v, idx2_v, idx3_v, rows1, rows2, rows3, outv, maskv, sem):
        wid = lax.axis_index("s") * 2 + lax.axis_index("c")
        base0 = wid * PPW

        def chunk_body(ci, _):
            pbase = base0 + ci * C
            pltpu.sync_copy(ids1_hbm.at[pl.ds(pbase * L, IPC)], idx1_v)
            pltpu.sync_copy(ids2_hbm.at[pl.ds(pbase * L, IPC)], idx2_v)
            pltpu.sync_copy(ids3_hbm.at[pl.ds(pbase * L, IPC)], idx3_v)
            pltpu.sync_copy(mask_hbm.at[pl.ds(pbase, C)], maskv)
            cps = []
            for g in range(NG):
                sl = pl.ds(g * G, G)
                cps.append(pltpu.async_copy(t1.at[idx1_v.at[sl]], rows1.at[sl], sem))
                cps.append(pltpu.async_copy(t2.at[idx2_v.at[sl]], rows2.at[sl], sem))
                cps.append(pltpu.async_copy(t3.at[idx3_v.at[sl]], rows3.at[sl], sem))
            for cp in cps:
                cp.wait()

            def patch_body(i, carry):
                rbase = i * L

                def l_body(l, accs):
                    r = rbase + l
                    a0, a1, a2, a3 = accs
                    for rows in (rows1, rows2, rows3):
                        a0 = a0 + rows[r, pl.ds(0, 16)]
                        a1 = a1 + rows[r, pl.ds(16, 16)]
                        a2 = a2 + rows[r, pl.ds(32, 16)]
                        a3 = a3 + rows[r, pl.ds(48, 16)]
                    return (a0, a1, a2, a3)

                z = jnp.zeros((16,), jnp.float32)
                a0, a1, a2, a3 = lax.fori_loop(0, L, l_body, (z, z, z, z))
                m = lax.gather(
                    maskv[...], jnp.full((16, 1), i, jnp.int32),
                    dimension_numbers=lax.GatherDimensionNumbers(
                        offset_dims=(), collapsed_slice_dims=(0,),
                        start_index_map=(0,)),
                    slice_sizes=(1,),
                    mode=lax.GatherScatterMode.PROMISE_IN_BOUNDS)
                outv[i, pl.ds(0, 16)] = a0 * m
                outv[i, pl.ds(16, 16)] = a1 * m
                outv[i, pl.ds(32, 16)] = a2 * m
                outv[i, pl.ds(48, 16)] = a3 * m
                return 0

            lax.fori_loop(0, C, patch_body, 0)
            pltpu.sync_copy(outv, out_hbm.at[pl.ds(pbase, C)])
            return 0

        lax.fori_loop(0, NCHUNK, chunk_body, 0)

    return k(ids1, ids2, ids3, maskf, emb_1, emb_2, emb_3)


def kernel(ids_1, ids_2, ids_3, patch_mask, emb_1, emb_2, emb_3):
    ids1 = ids_1.reshape(BP * L)
    ids2 = ids_2.reshape(BP * L)
    ids3 = ids_3.reshape(BP * L)
    maskf = patch_mask.reshape(BP).astype(jnp.float32)
    out = _sc_embed(ids1, ids2, ids3, maskf, emb_1, emb_2, emb_3)
    return out.reshape(B, P, D)
